# baseline (speedup 1.0000x reference)
import functools

import jax
import jax.numpy as jnp
from jax import lax
from jax.experimental import pallas as pl
from jax.experimental.pallas import tpu as pltpu
from jax.experimental.pallas import tpu_sc as plsc

NC = 2
NS = 16
L = 16
NW = NC * NS

SUB = 8
SGRP = 64


def _fused_body(b_per_w, d,
                short_hbm, ids_hbm, table_hbm, alpha_hbm, out_hbm,
                ids_v, rows_v, short_a, short_b, alpha_v, sem, ssem):
    wid = lax.axis_index("s") * NC + lax.axis_index("c")
    base = wid * b_per_w

    pltpu.sync_copy(ids_hbm.at[pl.ds(base, b_per_w)], ids_v)
    pltpu.sync_copy(alpha_hbm, alpha_v)

    lanes = lax.iota(jnp.int32, L)

    def issue(g, carry):
        v = ids_v[pl.ds(g * L, L)]
        for l in range(L):
            row = jnp.max(jnp.where(lanes == l, v, 0))
            jb = (g * L + l) // SUB
            pltpu.async_copy(
                table_hbm.at[pl.ds(row, 1)],
                rows_v.at[jb, pl.ds(l % SUB, 1)], sem)
        return carry

    lax.fori_loop(0, b_per_w // L, issue, 0)

    sbufs = [short_a, short_b]
    n_grp = b_per_w // SGRP
    c0 = pltpu.async_copy(
        short_hbm.at[pl.ds(base // SUB, SGRP // SUB)], sbufs[0], ssem)
    pltpu.make_async_copy(out_hbm.at[pl.ds(0, b_per_w // SUB)],
                          rows_v, sem).wait()

    a = 1.0 / (1.0 + jnp.exp(-alpha_v[...]))
    om_a = 1.0 - a

    pending = c0
    for grp in range(n_grp):
        pending.wait()
        if grp + 1 < n_grp:
            pending = pltpu.async_copy(
                short_hbm.at[pl.ds((base + (grp + 1) * SGRP) // SUB,
                                   SGRP // SUB)],
                sbufs[(grp + 1) % 2], ssem)
        sbuf = sbufs[grp % 2]

        def body(j, carry, grp=grp, sbuf=sbuf):
            jb = grp * (SGRP // SUB) + j // SUB
            sb = j // SUB
            js = lax.rem(j, SUB)
            for dj in range(d // L):
                sl = pl.ds(dj * L, L)
                r = rows_v[jb, js, sl]
                s = sbuf[sb, js, sl]
                rows_v[jb, js, sl] = a * s + om_a * r
            return carry

        lax.fori_loop(0, SGRP, body, 0, unroll=2)

    pltpu.sync_copy(rows_v, out_hbm.at[pl.ds(base // SUB, b_per_w // SUB)])


def kernel(short_term_interest, user_ids, long_term_emb, alpha):
    b, d = short_term_interest.shape
    b_per_w = b // NW

    ids = user_ids.astype(jnp.int32)
    alpha_vec = jnp.broadcast_to(jnp.asarray(alpha, jnp.float32).reshape(()), (L,))
    short_t = short_term_interest.reshape(b // SUB, SUB, d)

    mesh = plsc.VectorSubcoreMesh(core_axis_name="c", subcore_axis_name="s",
                                  num_cores=NC, num_subcores=NS)
    fused = functools.partial(
        pl.kernel,
        out_type=jax.ShapeDtypeStruct((b // SUB, SUB, d), jnp.float32),
        mesh=mesh,
        scratch_types=[
            pltpu.VMEM((b_per_w,), jnp.int32),
            pltpu.VMEM((b_per_w // SUB, SUB, d), jnp.float32),
            pltpu.VMEM((SGRP // SUB, SUB, d), jnp.float32),
            pltpu.VMEM((SGRP // SUB, SUB, d), jnp.float32),
            pltpu.VMEM((L,), jnp.float32),
            pltpu.SemaphoreType.DMA,
            pltpu.SemaphoreType.DMA,
        ],
        compiler_params=pltpu.CompilerParams(needs_layout_passes=False),
    )(functools.partial(_fused_body, b_per_w, d))
    out_t = fused(short_t, ids, long_term_emb, alpha_vec)
    return out_t.reshape(b, d)


# speedup vs baseline: 1.0001x; 1.0001x over previous
import functools

import jax
import jax.numpy as jnp
from jax import lax
from jax.experimental import pallas as pl
from jax.experimental.pallas import tpu as pltpu
from jax.experimental.pallas import tpu_sc as plsc

NC = 2
NS = 16
L = 16
NW = NC * NS

SUB = 8
SGRP = 64


def _fused_body(b_per_w, d,
                short_hbm, ids_hbm, table_hbm, alpha_hbm, out_hbm,
                ids_v, rows_v, short_a, short_b, alpha_v, sem, ssem):
    wid = lax.axis_index("s") * NC + lax.axis_index("c")
    base = wid * b_per_w

    pltpu.sync_copy(ids_hbm.at[pl.ds(base, b_per_w)], ids_v)
    pltpu.sync_copy(alpha_hbm, alpha_v)

    def issue(g, carry):
        v = ids_v[pl.ds(g * L, L)]
        for l in range(L):
            row = v[l]
            jb = (g * L + l) // SUB
            pltpu.async_copy(
                table_hbm.at[pl.ds(row, 1)],
                rows_v.at[jb, pl.ds(l % SUB, 1)], sem)
        return carry

    lax.fori_loop(0, b_per_w // L, issue, 0)

    sbufs = [short_a, short_b]
    n_grp = b_per_w // SGRP
    c0 = pltpu.async_copy(
        short_hbm.at[pl.ds(base // SUB, SGRP // SUB)], sbufs[0], ssem)
    pltpu.make_async_copy(out_hbm.at[pl.ds(0, b_per_w // SUB)],
                          rows_v, sem).wait()

    a = 1.0 / (1.0 + jnp.exp(-alpha_v[...]))
    om_a = 1.0 - a

    pending = c0
    for grp in range(n_grp):
        pending.wait()
        if grp + 1 < n_grp:
            pending = pltpu.async_copy(
                short_hbm.at[pl.ds((base + (grp + 1) * SGRP) // SUB,
                                   SGRP // SUB)],
                sbufs[(grp + 1) % 2], ssem)
        sbuf = sbufs[grp % 2]

        def body(j, carry, grp=grp, sbuf=sbuf):
            jb = grp * (SGRP // SUB) + j // SUB
            sb = j // SUB
            js = lax.rem(j, SUB)
            for dj in range(d // L):
                sl = pl.ds(dj * L, L)
                r = rows_v[jb, js, sl]
                s = sbuf[sb, js, sl]
                rows_v[jb, js, sl] = a * s + om_a * r
            return carry

        lax.fori_loop(0, SGRP, body, 0, unroll=2)

    pltpu.sync_copy(rows_v, out_hbm.at[pl.ds(base // SUB, b_per_w // SUB)])


def kernel(short_term_interest, user_ids, long_term_emb, alpha):
    b, d = short_term_interest.shape
    b_per_w = b // NW

    ids = user_ids.astype(jnp.int32)
    alpha_vec = jnp.broadcast_to(jnp.asarray(alpha, jnp.float32).reshape(()), (L,))
    short_t = short_term_interest.reshape(b // SUB, SUB, d)

    mesh = plsc.VectorSubcoreMesh(core_axis_name="c", subcore_axis_name="s",
                                  num_cores=NC, num_subcores=NS)
    fused = functools.partial(
        pl.kernel,
        out_type=jax.ShapeDtypeStruct((b // SUB, SUB, d), jnp.float32),
        mesh=mesh,
        scratch_types=[
            pltpu.VMEM((b_per_w,), jnp.int32),
            pltpu.VMEM((b_per_w // SUB, SUB, d), jnp.float32),
            pltpu.VMEM((SGRP // SUB, SUB, d), jnp.float32),
            pltpu.VMEM((SGRP // SUB, SUB, d), jnp.float32),
            pltpu.VMEM((L,), jnp.float32),
            pltpu.SemaphoreType.DMA,
            pltpu.SemaphoreType.DMA,
        ],
    )(functools.partial(_fused_body, b_per_w, d))
    out_t = fused(short_t, ids, long_term_emb, alpha_vec)
    return out_t.reshape(b, d)


# speedup vs baseline: 1.4522x; 1.4520x over previous
import functools

import jax
import jax.numpy as jnp
from jax import lax
from jax.experimental import pallas as pl
from jax.experimental.pallas import tpu as pltpu
from jax.experimental.pallas import tpu_sc as plsc

NC = 2
NS = 16
L = 16
NW = NC * NS

SUB = 8
SPAN = 512


def _sweep_body(b_per_w, d, nrows,
                sorted_hbm, tableT_hbm, g_hbm,
                sid_v, span_v, tail_v, rowbuf, sem):
    tail_start = (nrows // 128) * 128
    tail_w = nrows - tail_start
    max_start = (nrows - SPAN) // 128 * 128

    wid = lax.axis_index("s") * NC + lax.axis_index("c")
    base = wid * b_per_w

    pltpu.sync_copy(sorted_hbm.at[pl.ds(base, b_per_w)], sid_v)
    if tail_w:
        pltpu.sync_copy(tableT_hbm.at[:, pl.ds(tail_start, tail_w)], tail_v)

    lanes = lax.iota(jnp.int32, L)

    def group(gg, carry):
        v = sid_v[pl.ds(gg * L, L)]
        v0 = v[0]
        v15 = v[15]
        start0 = (v0 // 128) * 128
        npass = (v15 - start0) // SPAN + 1
        rows = gg * L + lanes

        def span_pass(t, c2):
            start = jnp.minimum(start0 + t * SPAN, max_start)
            pltpu.sync_copy(tableT_hbm.at[:, pl.ds(start, SPAN)], span_v)
            idx = v - start
            active = (idx >= 0) & (idx < SPAN)
            idxc = jnp.clip(idx, 0, SPAN - 1)
            rows_b = rows // SUB
            rows_s = lax.rem(rows, SUB)
            for c in range(d):
                cvec = jnp.full((L,), c, jnp.int32)
                vals = plsc.load_gather(span_v, [cvec, idxc])
                plsc.store_scatter(rowbuf, [rows_b, rows_s, cvec], vals,
                                   mask=active)
            return c2

        lax.fori_loop(0, npass, span_pass, 0)

        if tail_w:
            @pl.when(v15 >= tail_start)
            def _():
                idx_t = v - tail_start
                active_t = idx_t >= 0
                idxc_t = jnp.clip(idx_t, 0, tail_w - 1)
                rows_b = rows // SUB
                rows_s = lax.rem(rows, SUB)
                for c in range(d):
                    cvec = jnp.full((L,), c, jnp.int32)
                    vals = plsc.load_gather(tail_v, [cvec, idxc_t])
                    plsc.store_scatter(rowbuf, [rows_b, rows_s, cvec], vals,
                                       mask=active_t)
        return carry

    lax.fori_loop(0, b_per_w // L, group, 0)

    pltpu.sync_copy(rowbuf, g_hbm.at[pl.ds(base // SUB, b_per_w // SUB)])


def _lerp_body(b_per_w, d,
               short_hbm, inv_hbm, g_hbm, alpha_hbm, out_hbm,
               inv_v, rows_v, short_v, alpha_v, sem, ssem):
    half = b_per_w // 2
    wid = lax.axis_index("s") * NC + lax.axis_index("c")
    base = wid * b_per_w

    pltpu.sync_copy(inv_hbm.at[pl.ds(base, b_per_w)], inv_v)
    pltpu.sync_copy(alpha_hbm, alpha_v)

    a = 1.0 / (1.0 + jnp.exp(-alpha_v[...]))
    om_a = 1.0 - a

    for h in range(2):
        hbase = base + h * half

        def issue(g, carry, h=h):
            v = inv_v[pl.ds(h * half + g * L, L)]
            for l in range(L):
                sp = v[l]
                spb = sp // SUB
                spr = lax.rem(sp, SUB)
                jb = (g * L + l) // SUB
                pltpu.async_copy(g_hbm.at[spb, pl.ds(spr, 1)],
                                 rows_v.at[jb, pl.ds(l % SUB, 1)], sem)
            return carry

        lax.fori_loop(0, half // L, issue, 0)

        c0 = pltpu.async_copy(
            short_hbm.at[pl.ds(hbase // SUB, half // SUB)], short_v, ssem)
        pltpu.make_async_copy(out_hbm.at[pl.ds(0, half // SUB)],
                              rows_v, sem).wait()
        c0.wait()

        def body(j, carry):
            jb = j // SUB
            js = lax.rem(j, SUB)
            for dj in range(d // L):
                sl = pl.ds(dj * L, L)
                r = rows_v[jb, js, sl]
                s = short_v[jb, js, sl]
                short_v[jb, js, sl] = a * s + om_a * r
            return carry

        lax.fori_loop(0, half, body, 0, unroll=2)

        pltpu.sync_copy(short_v,
                        out_hbm.at[pl.ds(hbase // SUB, half // SUB)])


def kernel(short_term_interest, user_ids, long_term_emb, alpha):
    b, d = short_term_interest.shape
    nrows = long_term_emb.shape[0]
    b_per_w = b // NW

    ids = user_ids.astype(jnp.int32)
    order = jnp.argsort(ids).astype(jnp.int32)
    sorted_ids = jnp.take(ids, order, axis=0)
    inv = jnp.zeros((b,), jnp.int32).at[order].set(
        jnp.arange(b, dtype=jnp.int32))
    alpha_vec = jnp.broadcast_to(jnp.asarray(alpha, jnp.float32).reshape(()), (L,))
    tableT = jnp.transpose(long_term_emb)
    short_t = short_term_interest.reshape(b // SUB, SUB, d)
    tail_w = nrows - (nrows // 128) * 128

    mesh = plsc.VectorSubcoreMesh(core_axis_name="c", subcore_axis_name="s",
                                  num_cores=NC, num_subcores=NS)

    sweep = functools.partial(
        pl.kernel,
        out_type=jax.ShapeDtypeStruct((b // SUB, SUB, d), jnp.float32),
        mesh=mesh,
        scratch_types=[
            pltpu.VMEM((b_per_w,), jnp.int32),
            pltpu.VMEM((d, SPAN), jnp.float32),
            pltpu.VMEM((d, max(tail_w, 1)), jnp.float32),
            pltpu.VMEM((b_per_w // SUB, SUB, d), jnp.float32),
            pltpu.SemaphoreType.DMA,
        ],
        compiler_params=pltpu.CompilerParams(needs_layout_passes=False),
    )(functools.partial(_sweep_body, b_per_w, d, nrows))
    gathered = sweep(sorted_ids, tableT)

    lerp = functools.partial(
        pl.kernel,
        out_type=jax.ShapeDtypeStruct((b // SUB, SUB, d), jnp.float32),
        mesh=mesh,
        scratch_types=[
            pltpu.VMEM((b_per_w,), jnp.int32),
            pltpu.VMEM((b_per_w // 2 // SUB, SUB, d), jnp.float32),
            pltpu.VMEM((b_per_w // 2 // SUB, SUB, d), jnp.float32),
            pltpu.VMEM((L,), jnp.float32),
            pltpu.SemaphoreType.DMA,
            pltpu.SemaphoreType.DMA,
        ],
    )(functools.partial(_lerp_body, b_per_w, d))
    out_t = lerp(short_t, inv, gathered, alpha_vec)
    return out_t.reshape(b, d)
